# baseline (speedup 1.0000x reference)
import functools

import jax
import jax.numpy as jnp
import numpy as np
from jax import lax
from jax.experimental import pallas as pl
from jax.experimental.pallas import tpu as pltpu
from jax.experimental.pallas import tpu_sc as plsc

N = 50000
E = 800000
S_IN = 64
V_IN = 8
HS = 64
HV = 32
NUM_LAYERS = 3
NB = 32
CUTOFF = 5.0
C = 64
NSH = 9
DIM_IN = S_IN + 3 * V_IN
DIM_H = HS + 3 * HV

NC = 2
NS = 16
NW = NC * NS

NPASS = 3
NTHIRD = 16800
DUMP = 600
ACC_ROWS = NTHIRD + DUMP
CHALF = DIM_H // 2
SK = 80
EPT = E // NS

GK = 1000
EPW = E // NW

_mesh = functools.partial(
    plsc.VectorSubcoreMesh, core_axis_name="c", subcore_axis_name="s",
    num_cores=NC)


IW = 125
IR = GK // IW


def _make_row_gather_kernel(tables_and_outs, d):
    def body(*refs):
        npair = tables_and_outs
        tabs = refs[0:npair]
        idx2d = refs[npair:npair + npair]
        outs = refs[2 * npair:3 * npair]
        idx_v = refs[3 * npair]
        rows_v = refs[3 * npair + 1:3 * npair + 1 + npair]
        sem = refs[-1]
        wid = lax.axis_index("c") * NS + lax.axis_index("s")
        rbase = wid * (EPW // IW)

        def chunk(k, carry):
            off = wid * EPW + k * GK
            for t in range(npair):
                pltpu.sync_copy(idx2d[t].at[pl.ds(rbase + k * IR, IR)], idx_v)
                cps = [
                    pltpu.async_copy(
                        tabs[t].at[idx_v.at[q]],
                        rows_v[t].at[pl.ds(q * IW, IW)], sem)
                    for q in range(IR)
                ]
                for cp in cps:
                    cp.wait()
                pltpu.sync_copy(rows_v[t], outs[t].at[pl.ds(off, GK)])
            return carry

        lax.fori_loop(0, EPW // GK, chunk, 0)

    return body


def _pos_gather(pos16, src2d, dst2d):
    return pl.kernel(
        _make_row_gather_kernel(2, 16),
        out_type=[
            jax.ShapeDtypeStruct((E, 16), jnp.float32),
            jax.ShapeDtypeStruct((E, 16), jnp.float32),
        ],
        mesh=_mesh(),
        compiler_params=pltpu.CompilerParams(use_tc_tiling_on_sc=False),
        scratch_types=[
            pltpu.VMEM((IR, IW), jnp.int32),
            pltpu.VMEM((GK, 16), jnp.float32),
            pltpu.VMEM((GK, 16), jnp.float32),
            pltpu.SemaphoreType.DMA,
        ],
    )(pos16, pos16, src2d, dst2d)


GK2 = 200
IW2 = 100
IR2 = GK2 // IW2
HCHUNKS = EPW // GK2


def _h_gather_kernel(h, src2d, out, ix0, ix1, r0, r1, sem0, sem1):
    wid = lax.axis_index("c") * NS + lax.axis_index("s")
    base = wid * EPW
    rbase = wid * (EPW // IW2)
    bufs = ((ix0, r0, sem0), (ix1, r1, sem1))

    def fire(c, b):
        ix, rv, sem = bufs[b]
        pltpu.sync_copy(src2d.at[pl.ds(rbase + c * IR2, IR2)], ix)
        for q in range(IR2):
            pltpu.async_copy(h.at[ix.at[q]], rv.at[pl.ds(q * IW2, IW2)], sem)

    def finish(c, b):
        ix, rv, sem = bufs[b]
        for q in range(IR2):
            pltpu.make_async_copy(
                h.at[ix.at[q]], rv.at[pl.ds(q * IW2, IW2)], sem).wait()
        pltpu.sync_copy(rv, out.at[pl.ds(base + c * GK2, GK2)])

    fire(0, 0)

    def pair(k2, carry):
        c0 = 2 * k2
        c1 = c0 + 1
        fire(c1, 1)
        finish(c0, 0)
        @pl.when(c0 + 2 < HCHUNKS)
        def _():
            fire(c0 + 2, 0)
        finish(c1, 1)
        return carry

    lax.fori_loop(0, HCHUNKS // 2, pair, 0)
    finish(HCHUNKS - 1, 0)


def _h_gather(h, src2d_h):
    return pl.kernel(
        _h_gather_kernel,
        out_type=jax.ShapeDtypeStruct((E, C), jnp.float32),
        mesh=_mesh(),
        compiler_params=pltpu.CompilerParams(use_tc_tiling_on_sc=False),
        scratch_types=[
            pltpu.VMEM((IR2, IW2), jnp.int32),
            pltpu.VMEM((IR2, IW2), jnp.int32),
            pltpu.VMEM((GK2, C), jnp.float32),
            pltpu.VMEM((GK2, C), jnp.float32),
            pltpu.SemaphoreType.DMA,
            pltpu.SemaphoreType.DMA,
        ],
    )(h, src2d_h)


SR = 1
SW = SK // SR


NCHUNK = EPT // SK


def _scatter_kernel(y, dst1d, out, acc, yv0, yv1, dv0, dv1, iv0, iv1, zv,
                    lsem0, lsem1, ssem0, ssem1):
    c = lax.axis_index("c")
    tid = lax.axis_index("s")
    lanes = lax.iota(jnp.int32, 16)
    colo = c * CHALF
    bufs = ((yv0, dv0, iv0, lsem0, ssem0), (yv1, dv1, iv1, lsem1, ssem1))

    def issue_load(eoff, b):
        yv, dv, _, lsem, _ = bufs[b]
        pltpu.async_copy(y.at[pl.ds(eoff, SK), pl.ds(colo, CHALF)], yv, lsem)
        pltpu.async_copy(dst1d.at[pl.ds(eoff, SK)], dv, lsem)

    def wait_load(eoff, b):
        yv, dv, _, lsem, _ = bufs[b]
        pltpu.make_async_copy(
            y.at[pl.ds(eoff, SK), pl.ds(colo, CHALF)], yv, lsem).wait()
        pltpu.make_async_copy(dst1d.at[pl.ds(eoff, SK)], dv, lsem).wait()

    def drain_scat(b):
        yv, _, iv, _, ssem = bufs[b]
        for q in range(SR):
            pltpu.make_async_copy(
                yv.at[pl.ds(q * SW, SW)], acc.at[iv.at[q]], ssem).wait()

    def zrow(rr, carry):
        for cg in range(CHALF // 16):
            zv[rr, pl.ds(cg * 16, 16)] = jnp.zeros((16,), jnp.float32)
        return carry
    lax.fori_loop(0, 64, zrow, 0)

    for p in range(NPASS):
        nbase = p * NTHIRD
        for zc in range(18):
            zro = tid * 1088 + zc * 64
            @pl.when(zro + 64 <= ACC_ROWS)
            def _():
                pltpu.sync_copy(zv, acc.at[pl.ds(zro, 64)])
        plsc.subcore_barrier()

        ebase = tid * EPT

        def process(k, b):
            yv, dv, iv, _, ssem = bufs[b]
            for q in range(SR):
                for g in range(SW // 16):
                    o = q * SW + g * 16
                    d = dv[pl.ds(o, 16)]
                    inr = (d >= nbase) & (d < nbase + NTHIRD)
                    spread = lax.rem(lanes + (o + tid * 37), DUMP)
                    iv[q, pl.ds(g * 16, 16)] = jnp.where(
                        inr, d - nbase, NTHIRD + spread)
            for q in range(SR):
                pltpu.async_copy(
                    yv.at[pl.ds(q * SW, SW)], acc.at[iv.at[q]], ssem,
                    add=True)

        issue_load(ebase, 0)

        def pair(k2, carry):
            c0 = 2 * k2
            c1 = 2 * k2 + 1

            @pl.when(k2 > 0)
            def _():
                drain_scat(1)
            issue_load(ebase + c1 * SK, 1)
            wait_load(ebase + c0 * SK, 0)
            process(c0, 0)

            drain_scat(0)
            @pl.when(c1 + 1 < NCHUNK)
            def _():
                issue_load(ebase + (c1 + 1) * SK, 0)
            wait_load(ebase + c1 * SK, 1)
            process(c1, 1)
            return carry

        lax.fori_loop(0, NCHUNK // 2, pair, 0)
        wait_load(ebase + (NCHUNK - 1) * SK, 0)
        process(NCHUNK - 1, 0)
        drain_scat(0)
        drain_scat(1)
        plsc.subcore_barrier()

        rpt = NTHIRD // NS
        for dc, dlen in tuple((i * 80, 80) for i in range(13)) + ((1040, 10),):
            roff = tid * rpt + dc
            pltpu.sync_copy(acc.at[pl.ds(roff, dlen)], yv0.at[pl.ds(0, dlen)])
            pltpu.sync_copy(
                yv0.at[pl.ds(0, dlen)],
                out.at[pl.ds(p * NTHIRD + roff, dlen), pl.ds(colo, CHALF)])
        plsc.subcore_barrier()


def _scatter(y, dst):
    return pl.kernel(
        _scatter_kernel,
        out_type=jax.ShapeDtypeStruct((NPASS * NTHIRD, DIM_H), jnp.float32),
        mesh=_mesh(),
        compiler_params=pltpu.CompilerParams(use_tc_tiling_on_sc=False),
        scratch_types=[
            pltpu.VMEM_SHARED((ACC_ROWS, CHALF), jnp.float32),
            pltpu.VMEM((SK, CHALF), jnp.float32),
            pltpu.VMEM((SK, CHALF), jnp.float32),
            pltpu.VMEM((SK,), jnp.int32),
            pltpu.VMEM((SK,), jnp.int32),
            pltpu.VMEM((SR, SW), jnp.int32),
            pltpu.VMEM((SR, SW), jnp.int32),
            pltpu.VMEM((64, CHALF), jnp.float32),
            pltpu.SemaphoreType.DMA,
            pltpu.SemaphoreType.DMA,
            pltpu.SemaphoreType.DMA,
            pltpu.SemaphoreType.DMA,
        ],
    )(y, dst)


ECB = 6400


def _basis_kernel(ps_ref, pd_ref, sh_t_ref, cell_ref, o_ref):
    ps = ps_ref[...]
    pd = pd_ref[...]
    sh3 = sh_t_ref[...]
    cell0 = cell_ref[...]
    off = jnp.dot(cell0.T, sh3, preferred_element_type=jnp.float32)
    ev = pd - ps
    evt = ev[:, 0:3].T
    x = evt[0:1, :] + off[0:1, :]
    y = evt[1:2, :] + off[1:2, :]
    z = evt[2:3, :] + off[2:3, :]
    r2 = x * x + y * y + z * z
    r = jnp.sqrt(r2 + 1e-16)
    inv = 1.0 / jnp.maximum(r, 1e-8)
    dx = x * inv
    dy = y * inv
    dz = z * inv
    u = (r / CUTOFF) ** 2
    env = jnp.exp(1.0 - 1.0 / (1.0 - jnp.minimum(u, 0.99)))
    gate = jnp.where(r < CUTOFF, env, 0.0)
    c1 = np.float32(np.sqrt(3.0))
    c2 = np.float32(np.sqrt(15.0))
    c3 = np.float32(np.sqrt(5.0) / 2.0)
    rows = jnp.concatenate([
        r, gate,
        jnp.ones_like(dx),
        c1 * dy, c1 * dz, c1 * dx,
        c2 * dx * dy, c2 * dy * dz, c3 * (3.0 * dz * dz - 1.0),
        c2 * dx * dz, (c2 / 2.0) * (dx * dx - dy * dy),
        jnp.zeros((5, ECB), jnp.float32),
    ], axis=0)
    o_ref[...] = rows


def _basis(ps16, pd16, shifts_t, cell0):
    grid = (E // ECB,)
    return pl.pallas_call(
        _basis_kernel,
        grid=grid,
        in_specs=[
            pl.BlockSpec((ECB, 16), lambda i: (i, 0)),
            pl.BlockSpec((ECB, 16), lambda i: (i, 0)),
            pl.BlockSpec((3, ECB), lambda i: (0, i)),
            pl.BlockSpec((3, 3), lambda i: (0, 0)),
        ],
        out_specs=pl.BlockSpec((16, ECB), lambda i: (0, i)),
        out_shape=jax.ShapeDtypeStruct((16, E), jnp.float32),
    )(ps16, pd16, shifts_t, cell0)


NBLK = 2000


def _matmul_kernel(x_ref, w_ref, o_ref):
    o_ref[...] = jnp.dot(x_ref[...], w_ref[...],
                         preferred_element_type=jnp.float32)


def _node_matmul(x, w):
    n, d = x.shape
    dout = w.shape[1]
    return pl.pallas_call(
        _matmul_kernel,
        grid=(n // NBLK,),
        in_specs=[
            pl.BlockSpec((NBLK, d), lambda i: (i, 0)),
            pl.BlockSpec((d, dout), lambda i: (0, 0)),
        ],
        out_specs=pl.BlockSpec((NBLK, dout), lambda i: (i, 0)),
        out_shape=jax.ShapeDtypeStruct((n, dout), jnp.float32),
    )(x, w)


ECE = 3200


def _edge_kernel(b_ref, hs_ref, w1t_ref, b1_ref, w2t_ref, wout_ref, y_ref):
    b = b_ref[...]
    r = b[0:1, :]
    width = CUTOFF / NB
    centers = lax.broadcasted_iota(jnp.int32, (NB, 1), 0).astype(
        jnp.float32) * (CUTOFF / (NB - 1))
    diff = r - centers
    attr_t = jnp.exp(-(diff * diff) / (2.0 * width * width)) * b[1:2, :]
    pre_t = jnp.dot(w1t_ref[...], attr_t, preferred_element_type=jnp.float32)
    pre_t = jnp.maximum(pre_t + b1_ref[...], 0.0)
    r_t = jnp.dot(w2t_ref[...], pre_t, preferred_element_type=jnp.float32)
    hst = hs_ref[...].T
    m_t = hst * r_t * b[1:2, :]
    msh_t = jnp.concatenate(
        [m_t * b[j + 2:j + 3, :] for j in range(NSH)], axis=0)
    y = lax.dot_general(
        msh_t.astype(jnp.bfloat16), wout_ref[...].astype(jnp.bfloat16),
        (((0,), (0,)), ((), ())), preferred_element_type=jnp.float32)
    y_ref[...] = y


def _edge_stage(basis16, h_src, w1, b1, w2, wout):
    return pl.pallas_call(
        _edge_kernel,
        grid=(E // ECE,),
        in_specs=[
            pl.BlockSpec((16, ECE), lambda i: (0, i)),
            pl.BlockSpec((ECE, C), lambda i: (i, 0)),
            pl.BlockSpec((C, NB), lambda i: (0, 0)),
            pl.BlockSpec((C, 1), lambda i: (0, 0)),
            pl.BlockSpec((C, C), lambda i: (0, 0)),
            pl.BlockSpec((NSH * C, DIM_H), lambda i: (0, 0)),
        ],
        out_specs=pl.BlockSpec((ECE, DIM_H), lambda i: (i, 0)),
        out_shape=jax.ShapeDtypeStruct((E, DIM_H), jnp.float32),
    )(basis16, h_src, w1, b1, w2, wout)


def _selfadd_kernel(agg_ref, x_ref, w_ref, o_ref):
    v = agg_ref[...] + jnp.dot(x_ref[...], w_ref[...],
                               preferred_element_type=jnp.float32)
    o_ref[...] = jnp.nan_to_num(v)


def _selfadd(agg, x, w):
    d = x.shape[1]
    return pl.pallas_call(
        _selfadd_kernel,
        grid=(N // NBLK,),
        in_specs=[
            pl.BlockSpec((NBLK, DIM_H), lambda i: (i, 0)),
            pl.BlockSpec((NBLK, d), lambda i: (i, 0)),
            pl.BlockSpec((d, DIM_H), lambda i: (0, 0)),
        ],
        out_specs=pl.BlockSpec((NBLK, DIM_H), lambda i: (i, 0)),
        out_shape=jax.ShapeDtypeStruct((N, DIM_H), jnp.float32),
    )(agg, x, w)


def kernel(pos, shifts, cell, scalar_features, vector_features, params,
           edge_index, z):
    pos16 = jnp.pad(pos, ((0, 0), (0, 13)))
    src = edge_index[0]
    dst = edge_index[1]
    src2d = src.reshape(E // IW, IW)
    dst2d = dst.reshape(E // IW, IW)
    src2d_h = src.reshape(E // IW2, IW2)
    ps16, pd16 = _pos_gather(pos16, src2d, dst2d)
    basis16 = _basis(ps16, pd16, shifts.T, cell[0])
    sf = jnp.nan_to_num(scalar_features)
    vf = jnp.nan_to_num(vector_features)
    x = jnp.concatenate([sf, vf.reshape(vf.shape[0], -1)], axis=-1)

    for i in range(NUM_LAYERS):
        h = _node_matmul(x, params[f"W_pre_{i}"])
        h_src = _h_gather(h, src2d_h)
        y = _edge_stage(basis16, h_src, params[f"W1_{i}"].T,
                        params[f"b1_{i}"].reshape(C, 1), params[f"W2_{i}"].T,
                        params[f"W_out_{i}"])
        padded = _scatter(y, dst)
        x = _selfadd(padded[0:N], x, params[f"W_self_{i}"])

    scalar_out = x[:, :HS]
    vector_out = x[:, HS:].reshape(-1, HV, 3)
    return (x, scalar_out, vector_out)


# speedup vs baseline: 1.0068x; 1.0068x over previous
import functools

import jax
import jax.numpy as jnp
import numpy as np
from jax import lax
from jax.experimental import pallas as pl
from jax.experimental.pallas import tpu as pltpu
from jax.experimental.pallas import tpu_sc as plsc

N = 50000
E = 800000
S_IN = 64
V_IN = 8
HS = 64
HV = 32
NUM_LAYERS = 3
NB = 32
CUTOFF = 5.0
C = 64
NSH = 9
DIM_IN = S_IN + 3 * V_IN
DIM_H = HS + 3 * HV

NC = 2
NS = 16
NW = NC * NS

NPASS = 3
NTHIRD = 16800
DUMP = 600
ACC_ROWS = NTHIRD + DUMP
CHALF = DIM_H // 2
SK = 80
EPT = E // NS

GK = 1000
EPW = E // NW

_mesh = functools.partial(
    plsc.VectorSubcoreMesh, core_axis_name="c", subcore_axis_name="s",
    num_cores=NC)


IW = 125
IR = GK // IW


def _make_row_gather_kernel(tables_and_outs, d):
    def body(*refs):
        npair = tables_and_outs
        tabs = refs[0:npair]
        idx2d = refs[npair:npair + npair]
        outs = refs[2 * npair:3 * npair]
        idx_v = refs[3 * npair]
        rows_v = refs[3 * npair + 1:3 * npair + 1 + npair]
        sem = refs[-1]
        wid = lax.axis_index("c") * NS + lax.axis_index("s")
        rbase = wid * (EPW // IW)

        def chunk(k, carry):
            off = wid * EPW + k * GK
            for t in range(npair):
                pltpu.sync_copy(idx2d[t].at[pl.ds(rbase + k * IR, IR)], idx_v)
                cps = [
                    pltpu.async_copy(
                        tabs[t].at[idx_v.at[q]],
                        rows_v[t].at[pl.ds(q * IW, IW)], sem)
                    for q in range(IR)
                ]
                for cp in cps:
                    cp.wait()
                pltpu.sync_copy(rows_v[t], outs[t].at[pl.ds(off, GK)])
            return carry

        lax.fori_loop(0, EPW // GK, chunk, 0)

    return body


def _pos_gather(pos16, src2d, dst2d):
    return pl.kernel(
        _make_row_gather_kernel(2, 16),
        out_type=[
            jax.ShapeDtypeStruct((E, 16), jnp.float32),
            jax.ShapeDtypeStruct((E, 16), jnp.float32),
        ],
        mesh=_mesh(),
        compiler_params=pltpu.CompilerParams(use_tc_tiling_on_sc=False),
        scratch_types=[
            pltpu.VMEM((IR, IW), jnp.int32),
            pltpu.VMEM((GK, 16), jnp.float32),
            pltpu.VMEM((GK, 16), jnp.float32),
            pltpu.SemaphoreType.DMA,
        ],
    )(pos16, pos16, src2d, dst2d)


def _h_gather(h, src2d):
    return pl.kernel(
        _make_row_gather_kernel(1, C),
        out_type=jax.ShapeDtypeStruct((E, C), jnp.float32),
        mesh=_mesh(),
        compiler_params=pltpu.CompilerParams(use_tc_tiling_on_sc=False),
        scratch_types=[
            pltpu.VMEM((IR, IW), jnp.int32),
            pltpu.VMEM((GK, C), jnp.float32),
            pltpu.SemaphoreType.DMA,
        ],
    )(h, src2d)


SR = 1
SW = SK // SR


NCHUNK = EPT // SK


def _scatter_kernel(y, dst1d, out, acc, yv0, yv1, dv0, dv1, iv0, iv1, zv,
                    lsem0, lsem1, ssem0, ssem1):
    c = lax.axis_index("c")
    tid = lax.axis_index("s")
    lanes = lax.iota(jnp.int32, 16)
    colo = c * CHALF
    bufs = ((yv0, dv0, iv0, lsem0, ssem0), (yv1, dv1, iv1, lsem1, ssem1))

    def issue_load(eoff, b):
        yv, dv, _, lsem, _ = bufs[b]
        pltpu.async_copy(y.at[pl.ds(eoff, SK), pl.ds(colo, CHALF)], yv, lsem)
        pltpu.async_copy(dst1d.at[pl.ds(eoff, SK)], dv, lsem)

    def wait_load(eoff, b):
        yv, dv, _, lsem, _ = bufs[b]
        pltpu.make_async_copy(
            y.at[pl.ds(eoff, SK), pl.ds(colo, CHALF)], yv, lsem).wait()
        pltpu.make_async_copy(dst1d.at[pl.ds(eoff, SK)], dv, lsem).wait()

    def drain_scat(b):
        yv, _, iv, _, ssem = bufs[b]
        for q in range(SR):
            pltpu.make_async_copy(
                yv.at[pl.ds(q * SW, SW)], acc.at[iv.at[q]], ssem).wait()

    def zrow(rr, carry):
        for cg in range(CHALF // 16):
            zv[rr, pl.ds(cg * 16, 16)] = jnp.zeros((16,), jnp.float32)
        return carry
    lax.fori_loop(0, 64, zrow, 0)

    for p in range(NPASS):
        nbase = p * NTHIRD
        for zc in range(18):
            zro = tid * 1088 + zc * 64
            @pl.when(zro + 64 <= ACC_ROWS)
            def _():
                pltpu.sync_copy(zv, acc.at[pl.ds(zro, 64)])
        plsc.subcore_barrier()

        ebase = tid * EPT

        def process(k, b):
            yv, dv, iv, _, ssem = bufs[b]
            for q in range(SR):
                for g in range(SW // 16):
                    o = q * SW + g * 16
                    d = dv[pl.ds(o, 16)]
                    inr = (d >= nbase) & (d < nbase + NTHIRD)
                    spread = lax.rem(lanes + (o + tid * 37), DUMP)
                    iv[q, pl.ds(g * 16, 16)] = jnp.where(
                        inr, d - nbase, NTHIRD + spread)
            for q in range(SR):
                pltpu.async_copy(
                    yv.at[pl.ds(q * SW, SW)], acc.at[iv.at[q]], ssem,
                    add=True)

        issue_load(ebase, 0)

        def pair(k2, carry):
            c0 = 2 * k2
            c1 = 2 * k2 + 1

            @pl.when(k2 > 0)
            def _():
                drain_scat(1)
            issue_load(ebase + c1 * SK, 1)
            wait_load(ebase + c0 * SK, 0)
            process(c0, 0)

            drain_scat(0)
            @pl.when(c1 + 1 < NCHUNK)
            def _():
                issue_load(ebase + (c1 + 1) * SK, 0)
            wait_load(ebase + c1 * SK, 1)
            process(c1, 1)
            return carry

        lax.fori_loop(0, NCHUNK // 2, pair, 0)
        wait_load(ebase + (NCHUNK - 1) * SK, 0)
        process(NCHUNK - 1, 0)
        drain_scat(0)
        drain_scat(1)
        plsc.subcore_barrier()

        rpt = NTHIRD // NS
        for dc, dlen in tuple((i * 80, 80) for i in range(13)) + ((1040, 10),):
            roff = tid * rpt + dc
            pltpu.sync_copy(acc.at[pl.ds(roff, dlen)], yv0.at[pl.ds(0, dlen)])
            pltpu.sync_copy(
                yv0.at[pl.ds(0, dlen)],
                out.at[pl.ds(p * NTHIRD + roff, dlen), pl.ds(colo, CHALF)])
        plsc.subcore_barrier()


def _scatter(y, dst):
    return pl.kernel(
        _scatter_kernel,
        out_type=jax.ShapeDtypeStruct((NPASS * NTHIRD, DIM_H), jnp.float32),
        mesh=_mesh(),
        compiler_params=pltpu.CompilerParams(use_tc_tiling_on_sc=False),
        scratch_types=[
            pltpu.VMEM_SHARED((ACC_ROWS, CHALF), jnp.float32),
            pltpu.VMEM((SK, CHALF), jnp.float32),
            pltpu.VMEM((SK, CHALF), jnp.float32),
            pltpu.VMEM((SK,), jnp.int32),
            pltpu.VMEM((SK,), jnp.int32),
            pltpu.VMEM((SR, SW), jnp.int32),
            pltpu.VMEM((SR, SW), jnp.int32),
            pltpu.VMEM((64, CHALF), jnp.float32),
            pltpu.SemaphoreType.DMA,
            pltpu.SemaphoreType.DMA,
            pltpu.SemaphoreType.DMA,
            pltpu.SemaphoreType.DMA,
        ],
    )(y, dst)


ECB = 6400


def _basis_kernel(ps_ref, pd_ref, sh_t_ref, cell_ref, o_ref):
    ps = ps_ref[...]
    pd = pd_ref[...]
    sh3 = sh_t_ref[...]
    cell0 = cell_ref[...]
    off = jnp.dot(cell0.T, sh3, preferred_element_type=jnp.float32)
    ev = pd - ps
    evt = ev[:, 0:3].T
    x = evt[0:1, :] + off[0:1, :]
    y = evt[1:2, :] + off[1:2, :]
    z = evt[2:3, :] + off[2:3, :]
    r2 = x * x + y * y + z * z
    r = jnp.sqrt(r2 + 1e-16)
    inv = 1.0 / jnp.maximum(r, 1e-8)
    dx = x * inv
    dy = y * inv
    dz = z * inv
    u = (r / CUTOFF) ** 2
    env = jnp.exp(1.0 - 1.0 / (1.0 - jnp.minimum(u, 0.99)))
    gate = jnp.where(r < CUTOFF, env, 0.0)
    c1 = np.float32(np.sqrt(3.0))
    c2 = np.float32(np.sqrt(15.0))
    c3 = np.float32(np.sqrt(5.0) / 2.0)
    rows = jnp.concatenate([
        r, gate,
        jnp.ones_like(dx),
        c1 * dy, c1 * dz, c1 * dx,
        c2 * dx * dy, c2 * dy * dz, c3 * (3.0 * dz * dz - 1.0),
        c2 * dx * dz, (c2 / 2.0) * (dx * dx - dy * dy),
        jnp.zeros((5, ECB), jnp.float32),
    ], axis=0)
    o_ref[...] = rows


def _basis(ps16, pd16, shifts_t, cell0):
    grid = (E // ECB,)
    return pl.pallas_call(
        _basis_kernel,
        grid=grid,
        in_specs=[
            pl.BlockSpec((ECB, 16), lambda i: (i, 0)),
            pl.BlockSpec((ECB, 16), lambda i: (i, 0)),
            pl.BlockSpec((3, ECB), lambda i: (0, i)),
            pl.BlockSpec((3, 3), lambda i: (0, 0)),
        ],
        out_specs=pl.BlockSpec((16, ECB), lambda i: (0, i)),
        out_shape=jax.ShapeDtypeStruct((16, E), jnp.float32),
    )(ps16, pd16, shifts_t, cell0)


NBLK = 2000


def _matmul_kernel(x_ref, w_ref, o_ref):
    o_ref[...] = jnp.dot(x_ref[...], w_ref[...],
                         preferred_element_type=jnp.float32)


def _node_matmul(x, w):
    n, d = x.shape
    dout = w.shape[1]
    return pl.pallas_call(
        _matmul_kernel,
        grid=(n // NBLK,),
        in_specs=[
            pl.BlockSpec((NBLK, d), lambda i: (i, 0)),
            pl.BlockSpec((d, dout), lambda i: (0, 0)),
        ],
        out_specs=pl.BlockSpec((NBLK, dout), lambda i: (i, 0)),
        out_shape=jax.ShapeDtypeStruct((n, dout), jnp.float32),
    )(x, w)


ECE = 3200


def _edge_kernel(b_ref, hs_ref, w1t_ref, b1_ref, w2t_ref, wout_ref, y_ref):
    b = b_ref[...]
    r = b[0:1, :]
    width = CUTOFF / NB
    centers = lax.broadcasted_iota(jnp.int32, (NB, 1), 0).astype(
        jnp.float32) * (CUTOFF / (NB - 1))
    diff = r - centers
    attr_t = jnp.exp(-(diff * diff) / (2.0 * width * width)) * b[1:2, :]
    pre_t = jnp.dot(w1t_ref[...], attr_t, preferred_element_type=jnp.float32)
    pre_t = jnp.maximum(pre_t + b1_ref[...], 0.0)
    r_t = jnp.dot(w2t_ref[...], pre_t, preferred_element_type=jnp.float32)
    hst = hs_ref[...].T
    m_t = hst * r_t * b[1:2, :]
    msh_t = jnp.concatenate(
        [m_t * b[j + 2:j + 3, :] for j in range(NSH)], axis=0)
    y = lax.dot_general(
        msh_t.astype(jnp.bfloat16), wout_ref[...].astype(jnp.bfloat16),
        (((0,), (0,)), ((), ())), preferred_element_type=jnp.float32)
    y_ref[...] = y


def _edge_stage(basis16, h_src, w1, b1, w2, wout):
    return pl.pallas_call(
        _edge_kernel,
        grid=(E // ECE,),
        in_specs=[
            pl.BlockSpec((16, ECE), lambda i: (0, i)),
            pl.BlockSpec((ECE, C), lambda i: (i, 0)),
            pl.BlockSpec((C, NB), lambda i: (0, 0)),
            pl.BlockSpec((C, 1), lambda i: (0, 0)),
            pl.BlockSpec((C, C), lambda i: (0, 0)),
            pl.BlockSpec((NSH * C, DIM_H), lambda i: (0, 0)),
        ],
        out_specs=pl.BlockSpec((ECE, DIM_H), lambda i: (i, 0)),
        out_shape=jax.ShapeDtypeStruct((E, DIM_H), jnp.float32),
    )(basis16, h_src, w1, b1, w2, wout)


def _selfadd_kernel(agg_ref, x_ref, w_ref, o_ref):
    v = agg_ref[...] + jnp.dot(x_ref[...], w_ref[...],
                               preferred_element_type=jnp.float32)
    o_ref[...] = jnp.nan_to_num(v)


def _selfadd(agg, x, w):
    d = x.shape[1]
    return pl.pallas_call(
        _selfadd_kernel,
        grid=(N // NBLK,),
        in_specs=[
            pl.BlockSpec((NBLK, DIM_H), lambda i: (i, 0)),
            pl.BlockSpec((NBLK, d), lambda i: (i, 0)),
            pl.BlockSpec((d, DIM_H), lambda i: (0, 0)),
        ],
        out_specs=pl.BlockSpec((NBLK, DIM_H), lambda i: (i, 0)),
        out_shape=jax.ShapeDtypeStruct((N, DIM_H), jnp.float32),
    )(agg, x, w)


def kernel(pos, shifts, cell, scalar_features, vector_features, params,
           edge_index, z):
    pos16 = jnp.pad(pos, ((0, 0), (0, 13)))
    src = edge_index[0]
    dst = edge_index[1]
    src2d = src.reshape(E // IW, IW)
    dst2d = dst.reshape(E // IW, IW)
    ps16, pd16 = _pos_gather(pos16, src2d, dst2d)
    basis16 = _basis(ps16, pd16, shifts.T, cell[0])
    sf = jnp.nan_to_num(scalar_features)
    vf = jnp.nan_to_num(vector_features)
    x = jnp.concatenate([sf, vf.reshape(vf.shape[0], -1)], axis=-1)

    for i in range(NUM_LAYERS):
        h = _node_matmul(x, params[f"W_pre_{i}"])
        h_src = _h_gather(h, src2d)
        y = _edge_stage(basis16, h_src, params[f"W1_{i}"].T,
                        params[f"b1_{i}"].reshape(C, 1), params[f"W2_{i}"].T,
                        params[f"W_out_{i}"])
        padded = _scatter(y, dst)
        x = _selfadd(padded[0:N], x, params[f"W_self_{i}"])

    scalar_out = x[:, :HS]
    vector_out = x[:, HS:].reshape(-1, HV, 3)
    return (x, scalar_out, vector_out)
